# Initial kernel scaffold; baseline (speedup 1.0000x reference)
import jax
import jax.numpy as jnp
from jax.experimental import pallas as pl


def kernel(x, edge_index, batch, W1, b1, W2, b2, W3, b3):
    raise NotImplementedError("write your pallas kernel here")


# speedup vs baseline: 9.9533x; 9.9533x over previous
import jax
import jax.numpy as jnp
from jax import lax
from jax.experimental import pallas as pl
from jax.experimental.pallas import tpu as pltpu
from jax.experimental.pallas import tpu_sc as plsc

N = 10000
E = 320000
D = 128
H = 128

NS = 16
K = 128
NROWS = E // K
STRIPE = 624
TAIL = N - NS * STRIPE


def _sc_body(x_hbm, src_hbm, dst2d_hbm, agg_out, degp_out,
             src_idx, dst_idx, rows, deg_local, agg_sh, sem):
    sid = lax.axis_index("s")

    zv = jnp.zeros((16,), jnp.float32)

    def _zero_rows(i, _):
        rows[i // 8, pl.ds((i % 8) * 16, 16)] = zv
        return 0
    lax.fori_loop(0, K * D // 16, _zero_rows, 0)

    def _zero_deg(i, _):
        deg_local[0, pl.ds(i * 16, 16)] = zv
        return 0
    lax.fori_loop(0, N // 16, _zero_deg, 0)

    base = sid * STRIPE
    off = 0
    for sz in (128, 128, 128, 128, 112):
        pltpu.sync_copy(rows.at[pl.ds(0, sz)], agg_sh.at[pl.ds(base + off, sz)])
        off += sz

    @pl.when(sid == NS - 1)
    def _zero_tail():
        pltpu.sync_copy(rows.at[pl.ds(0, TAIL)],
                        agg_sh.at[pl.ds(NS * STRIPE, TAIL)])

    plsc.subcore_barrier()

    nrows_t = (NROWS - 1 - sid) // NS + 1
    ones_v = jnp.full((16,), 1.0, jnp.float32)
    zeros_i = jnp.zeros((16,), jnp.int32)

    def _edge_row(i, _):
        r = sid + i * NS
        pltpu.sync_copy(src_hbm.at[pl.ds(r * K, K)], src_idx)
        pltpu.sync_copy(dst2d_hbm.at[pl.ds(r, 1)], dst_idx)
        pltpu.async_copy(x_hbm.at[src_idx], rows, sem).wait()
        pltpu.sync_copy(rows, agg_sh.at[dst_idx.at[0]], add=True)
        for l in range(K // 16):
            d16 = dst_idx[0, pl.ds(l * 16, 16)]
            plsc.addupdate_scatter(deg_local, [zeros_i, d16], ones_v)
        return 0

    lax.fori_loop(0, nrows_t, _edge_row, 0)
    plsc.subcore_barrier()

    pltpu.sync_copy(agg_sh.at[pl.ds(base, STRIPE)],
                    agg_out.at[pl.ds(base, STRIPE)])

    @pl.when(sid == NS - 1)
    def _write_tail():
        pltpu.sync_copy(agg_sh.at[pl.ds(NS * STRIPE, TAIL)],
                        agg_out.at[pl.ds(NS * STRIPE, TAIL)])

    pltpu.sync_copy(deg_local.at[0], degp_out.at[pl.ds(sid * N, N)])


@jax.jit
def _sc_aggregate(x, src, dst2d):
    mesh = plsc.VectorSubcoreMesh(core_axis_name="c", subcore_axis_name="s",
                                  num_cores=1)
    f = pl.kernel(
        _sc_body,
        out_type=[
            jax.ShapeDtypeStruct((N, D), jnp.float32),
            jax.ShapeDtypeStruct((NS * N,), jnp.float32),
        ],
        mesh=mesh,
        compiler_params=pltpu.CompilerParams(needs_layout_passes=False),
        scratch_types=[
            pltpu.VMEM((K,), jnp.int32),
            pltpu.VMEM((1, K), jnp.int32),
            pltpu.VMEM((K, D), jnp.float32),
            pltpu.VMEM((1, N), jnp.float32),
            pltpu.VMEM_SHARED((N, D), jnp.float32),
            pltpu.SemaphoreType.DMA,
        ],
    )
    return f(x, src, dst2d)


def _tc_body(a, dp, w1, b1, w2, b2, w3, b3, o):
    deg = jnp.sum(dp[...], axis=1, keepdims=True)
    agg = a[...]
    acc = jnp.zeros_like(o)
    for w, b in ((w1, b1), (w2, b2), (w3, b3)):
        y = (jnp.dot(agg, w[...], preferred_element_type=jnp.float32)
             + deg * b[...])
        acc += jnp.maximum(y, 0.0)
    o[...] = acc / jnp.maximum(deg, 1.0)


@jax.jit
def _tc_dense(agg, degp, W1, b1, W2, b2, W3, b3):
    BR = 1000
    grid = (N // BR,)
    wspec = pl.BlockSpec((D, H), lambda i: (0, 0))
    bspec = pl.BlockSpec((1, H), lambda i: (0, 0))
    return pl.pallas_call(
        _tc_body,
        grid=grid,
        in_specs=[
            pl.BlockSpec((BR, D), lambda i: (i, 0)),
            pl.BlockSpec((BR, NS), lambda i: (i, 0)),
            wspec, bspec, wspec, bspec, wspec, bspec,
        ],
        out_specs=pl.BlockSpec((BR, H), lambda i: (i, 0)),
        out_shape=jax.ShapeDtypeStruct((N, H), jnp.float32),
    )(agg, degp, W1, b1, W2, b2, W3, b3)


def kernel(x, edge_index, batch, W1, b1, W2, b2, W3, b3):
    src = edge_index[0]
    dst2d = edge_index[1].reshape(NROWS, K)
    agg, degp = _sc_aggregate(x, src, dst2d)
    return _tc_dense(agg, degp.reshape(NS, N).T, W1, b1.reshape(1, H),
                     W2, b2.reshape(1, H), W3, b3.reshape(1, H))


# speedup vs baseline: 19.6511x; 1.9743x over previous
import jax
import jax.numpy as jnp
from jax import lax
from jax.experimental import pallas as pl
from jax.experimental.pallas import tpu as pltpu
from jax.experimental.pallas import tpu_sc as plsc

N = 10000
E = 320000
D = 128
H = 128

NS = 16
K = 128
NROWS = E // K
CMAX = NROWS // NS + 1
IB = 16
STRIPE = 624
TAIL = N - NS * STRIPE


def _sc_body(x_hbm, src_hbm, dst_hbm, agg_out, degp_out,
             src_blk, dst_blk, dst_stage, rows0, rows1, deg_local, agg_sh,
             sem0, sem1):
    sid = lax.axis_index("s")

    zv = jnp.zeros((16,), jnp.float32)

    def _zero_rows(i, _):
        rows0[i // 8, pl.ds((i % 8) * 16, 16)] = zv
        return 0
    lax.fori_loop(0, K * D // 16, _zero_rows, 0)

    def _zero_deg(i, _):
        deg_local[0, pl.ds(i * 16, 16)] = zv
        return 0
    lax.fori_loop(0, N // 16, _zero_deg, 0)

    base = sid * STRIPE
    off = 0
    for sz in (128, 128, 128, 128, 112):
        pltpu.sync_copy(rows0.at[pl.ds(0, sz)], agg_sh.at[pl.ds(base + off, sz)])
        off += sz

    @pl.when(sid == NS - 1)
    def _zero_tail():
        pltpu.sync_copy(rows0.at[pl.ds(0, TAIL)],
                        agg_sh.at[pl.ds(NS * STRIPE, TAIL)])

    plsc.subcore_barrier()

    c0 = (CMAX - 1) * sid + jnp.minimum(sid, NROWS - (CMAX - 1) * NS)
    nc = jnp.where(sid < NROWS - (CMAX - 1) * NS, CMAX, CMAX - 1)

    ones_v = jnp.full((16,), 1.0, jnp.float32)
    zeros_i = jnp.zeros((16,), jnp.int32)

    def _consume(j, rows, sem):
        for l in range(K // 16):
            d16 = dst_blk[pl.ds(j * K + l * 16, 16)]
            dst_stage[0, pl.ds(l * 16, 16)] = d16
            plsc.addupdate_scatter(deg_local, [zeros_i, d16], ones_v)
        pltpu.make_async_copy(x_hbm.at[src_blk.at[pl.ds(0, K)]], rows,
                              sem).wait()
        pltpu.sync_copy(rows, agg_sh.at[dst_stage.at[0]], add=True)

    def _fire(j, rows, sem):
        pltpu.async_copy(x_hbm.at[src_blk.at[pl.ds(j * K, K)]], rows, sem)

    def _block(b, _):
        bc = c0 + b * IB
        cnt = jnp.minimum(nc - b * IB, IB)

        @pl.when(cnt == IB)
        def _load_blk_full():
            pltpu.sync_copy(src_hbm.at[pl.ds(bc * K, IB * K)], src_blk)
            pltpu.sync_copy(dst_hbm.at[pl.ds(bc * K, IB * K)], dst_blk)

        @pl.when(cnt < IB)
        def _load_blk_part():
            @pl.when(cnt == CMAX % IB)
            def _t13():
                pltpu.sync_copy(src_hbm.at[pl.ds(bc * K, (CMAX % IB) * K)],
                                src_blk.at[pl.ds(0, (CMAX % IB) * K)])
                pltpu.sync_copy(dst_hbm.at[pl.ds(bc * K, (CMAX % IB) * K)],
                                dst_blk.at[pl.ds(0, (CMAX % IB) * K)])

            @pl.when(cnt == (CMAX - 1) % IB)
            def _t12():
                pltpu.sync_copy(
                    src_hbm.at[pl.ds(bc * K, ((CMAX - 1) % IB) * K)],
                    src_blk.at[pl.ds(0, ((CMAX - 1) % IB) * K)])
                pltpu.sync_copy(
                    dst_hbm.at[pl.ds(bc * K, ((CMAX - 1) % IB) * K)],
                    dst_blk.at[pl.ds(0, ((CMAX - 1) % IB) * K)])

        _fire(0, rows0, sem0)
        for jp in range(IB // 2):
            e = 2 * jp
            o = e + 1
            ne = e + 2

            @pl.when(o < cnt)
            def _fire_odd():
                _fire(o, rows1, sem1)

            if e == 0:
                _consume(e, rows0, sem0)
            else:
                @pl.when(e < cnt)
                def _consume_even():
                    _consume(e, rows0, sem0)

            if ne < IB:
                @pl.when(ne < cnt)
                def _fire_even():
                    _fire(ne, rows0, sem0)

            @pl.when(o < cnt)
            def _consume_odd():
                _consume(o, rows1, sem1)
        return 0

    lax.fori_loop(0, (nc + IB - 1) // IB, _block, 0)
    plsc.subcore_barrier()

    pltpu.sync_copy(agg_sh.at[pl.ds(base, STRIPE)],
                    agg_out.at[pl.ds(base, STRIPE)])

    @pl.when(sid == NS - 1)
    def _write_tail():
        pltpu.sync_copy(agg_sh.at[pl.ds(NS * STRIPE, TAIL)],
                        agg_out.at[pl.ds(NS * STRIPE, TAIL)])

    pltpu.sync_copy(deg_local.at[0], degp_out.at[pl.ds(sid * N, N)])


@jax.jit
def _sc_aggregate(x, src, dst):
    mesh = plsc.VectorSubcoreMesh(core_axis_name="c", subcore_axis_name="s",
                                  num_cores=1)
    f = pl.kernel(
        _sc_body,
        out_type=[
            jax.ShapeDtypeStruct((N, D), jnp.float32),
            jax.ShapeDtypeStruct((NS * N,), jnp.float32),
        ],
        mesh=mesh,
        compiler_params=pltpu.CompilerParams(needs_layout_passes=False),
        scratch_types=[
            pltpu.VMEM((IB * K,), jnp.int32),
            pltpu.VMEM((IB * K,), jnp.int32),
            pltpu.VMEM((1, K), jnp.int32),
            pltpu.VMEM((K, D), jnp.float32),
            pltpu.VMEM((K, D), jnp.float32),
            pltpu.VMEM((1, N), jnp.float32),
            pltpu.VMEM_SHARED((N, D), jnp.float32),
            pltpu.SemaphoreType.DMA,
            pltpu.SemaphoreType.DMA,
        ],
    )
    return f(x, src, dst)


def _tc_body(a, dp, w1, b1, w2, b2, w3, b3, o):
    deg = jnp.sum(dp[...], axis=1, keepdims=True)
    agg = a[...]
    acc = jnp.zeros_like(o)
    for w, b in ((w1, b1), (w2, b2), (w3, b3)):
        y = (jnp.dot(agg, w[...], preferred_element_type=jnp.float32)
             + deg * b[...])
        acc += jnp.maximum(y, 0.0)
    o[...] = acc / jnp.maximum(deg, 1.0)


@jax.jit
def _tc_dense(agg, degp, W1, b1, W2, b2, W3, b3):
    BR = 1000
    grid = (N // BR,)
    wspec = pl.BlockSpec((D, H), lambda i: (0, 0))
    bspec = pl.BlockSpec((1, H), lambda i: (0, 0))
    return pl.pallas_call(
        _tc_body,
        grid=grid,
        in_specs=[
            pl.BlockSpec((BR, D), lambda i: (i, 0)),
            pl.BlockSpec((BR, NS), lambda i: (i, 0)),
            wspec, bspec, wspec, bspec, wspec, bspec,
        ],
        out_specs=pl.BlockSpec((BR, H), lambda i: (i, 0)),
        out_shape=jax.ShapeDtypeStruct((N, H), jnp.float32),
    )(agg, degp, W1, b1, W2, b2, W3, b3)


def kernel(x, edge_index, batch, W1, b1, W2, b2, W3, b3):
    src = edge_index[0]
    dst = edge_index[1]
    agg, degp = _sc_aggregate(x, src, dst)
    return _tc_dense(agg, degp.reshape(NS, N).T, W1, b1.reshape(1, H),
                     W2, b2.reshape(1, H), W3, b3.reshape(1, H))


# speedup vs baseline: 30.4676x; 1.5504x over previous
import jax
import jax.numpy as jnp
from jax import lax
from jax.experimental import pallas as pl
from jax.experimental.pallas import tpu as pltpu
from jax.experimental.pallas import tpu_sc as plsc

N = 10000
E = 320000
D = 128
H = 128

NS = 16
K = 128
NROWS = E // K
NC = 2
NW = NC * NS
CMAX = NROWS // NW + 1
IB = 16
STRIPE = 624
TAIL = N - NS * STRIPE


def _sc_body(x_hbm, src_hbm, dst_hbm, agg_out, degp_out,
             src_blk, dst_blk, dst_stage, rows0, rows1, deg_local, agg_sh,
             sem0, sem1):
    cid = lax.axis_index("c")
    sid = lax.axis_index("s")
    wid = sid * NC + cid

    zv = jnp.zeros((16,), jnp.float32)

    def _zero_rows(i, _):
        rows0[i // 8, pl.ds((i % 8) * 16, 16)] = zv
        return 0
    lax.fori_loop(0, K * D // 16, _zero_rows, 0)

    def _zero_deg(i, _):
        deg_local[0, pl.ds(i * 16, 16)] = zv
        return 0
    lax.fori_loop(0, N // 16, _zero_deg, 0)

    base = sid * STRIPE
    off = 0
    for sz in (128, 128, 128, 128, 112):
        pltpu.sync_copy(rows0.at[pl.ds(0, sz)], agg_sh.at[pl.ds(base + off, sz)])
        off += sz

    @pl.when(sid == NS - 1)
    def _zero_tail():
        pltpu.sync_copy(rows0.at[pl.ds(0, TAIL)],
                        agg_sh.at[pl.ds(NS * STRIPE, TAIL)])

    plsc.subcore_barrier()

    c0 = (CMAX - 1) * wid + jnp.minimum(wid, NROWS - (CMAX - 1) * NW)
    nc = jnp.where(wid < NROWS - (CMAX - 1) * NW, CMAX, CMAX - 1)

    ones_v = jnp.full((16,), 1.0, jnp.float32)
    zeros_i = jnp.zeros((16,), jnp.int32)

    def _consume(j, rows, sem):
        for l in range(K // 16):
            d16 = dst_blk[pl.ds(j * K + l * 16, 16)]
            dst_stage[0, pl.ds(l * 16, 16)] = d16
            plsc.addupdate_scatter(deg_local, [zeros_i, d16], ones_v)
        pltpu.make_async_copy(x_hbm.at[src_blk.at[pl.ds(0, K)]], rows,
                              sem).wait()
        pltpu.sync_copy(rows, agg_sh.at[dst_stage.at[0]], add=True)

    def _fire(j, rows, sem):
        pltpu.async_copy(x_hbm.at[src_blk.at[pl.ds(j * K, K)]], rows, sem)

    def _block(b, _):
        bc = c0 + b * IB
        cnt = jnp.minimum(nc - b * IB, IB)

        @pl.when(cnt == IB)
        def _load_blk_full():
            pltpu.sync_copy(src_hbm.at[pl.ds(bc * K, IB * K)], src_blk)
            pltpu.sync_copy(dst_hbm.at[pl.ds(bc * K, IB * K)], dst_blk)

        @pl.when(cnt < IB)
        def _load_blk_part():
            @pl.when(cnt == CMAX % IB)
            def _t13():
                pltpu.sync_copy(src_hbm.at[pl.ds(bc * K, (CMAX % IB) * K)],
                                src_blk.at[pl.ds(0, (CMAX % IB) * K)])
                pltpu.sync_copy(dst_hbm.at[pl.ds(bc * K, (CMAX % IB) * K)],
                                dst_blk.at[pl.ds(0, (CMAX % IB) * K)])

            @pl.when(cnt == (CMAX - 1) % IB)
            def _t12():
                pltpu.sync_copy(
                    src_hbm.at[pl.ds(bc * K, ((CMAX - 1) % IB) * K)],
                    src_blk.at[pl.ds(0, ((CMAX - 1) % IB) * K)])
                pltpu.sync_copy(
                    dst_hbm.at[pl.ds(bc * K, ((CMAX - 1) % IB) * K)],
                    dst_blk.at[pl.ds(0, ((CMAX - 1) % IB) * K)])

        _fire(0, rows0, sem0)
        for jp in range(IB // 2):
            e = 2 * jp
            o = e + 1
            ne = e + 2

            @pl.when(o < cnt)
            def _fire_odd():
                _fire(o, rows1, sem1)

            if e == 0:
                _consume(e, rows0, sem0)
            else:
                @pl.when(e < cnt)
                def _consume_even():
                    _consume(e, rows0, sem0)

            if ne < IB:
                @pl.when(ne < cnt)
                def _fire_even():
                    _fire(ne, rows0, sem0)

            @pl.when(o < cnt)
            def _consume_odd():
                _consume(o, rows1, sem1)
        return 0

    lax.fori_loop(0, (nc + IB - 1) // IB, _block, 0)
    plsc.subcore_barrier()

    pltpu.sync_copy(agg_sh.at[pl.ds(base, STRIPE)],
                    agg_out.at[pl.ds(cid * N + base, STRIPE)])

    @pl.when(sid == NS - 1)
    def _write_tail():
        pltpu.sync_copy(agg_sh.at[pl.ds(NS * STRIPE, TAIL)],
                        agg_out.at[pl.ds(cid * N + NS * STRIPE, TAIL)])

    pltpu.sync_copy(deg_local.at[0], degp_out.at[pl.ds(wid * N, N)])


@jax.jit
def _sc_aggregate(x, src, dst):
    mesh = plsc.VectorSubcoreMesh(core_axis_name="c", subcore_axis_name="s")
    f = pl.kernel(
        _sc_body,
        out_type=[
            jax.ShapeDtypeStruct((NC * N, D), jnp.float32),
            jax.ShapeDtypeStruct((NW * N,), jnp.float32),
        ],
        mesh=mesh,
        compiler_params=pltpu.CompilerParams(needs_layout_passes=False),
        scratch_types=[
            pltpu.VMEM((IB * K,), jnp.int32),
            pltpu.VMEM((IB * K,), jnp.int32),
            pltpu.VMEM((1, K), jnp.int32),
            pltpu.VMEM((K, D), jnp.float32),
            pltpu.VMEM((K, D), jnp.float32),
            pltpu.VMEM((1, N), jnp.float32),
            pltpu.VMEM_SHARED((N, D), jnp.float32),
            pltpu.SemaphoreType.DMA,
            pltpu.SemaphoreType.DMA,
        ],
    )
    return f(x, src, dst)


def _tc_body(a0, a1, dp, w1, b1, w2, b2, w3, b3, o):
    deg = jnp.sum(dp[...], axis=1, keepdims=True)
    agg = a0[...] + a1[...]
    acc = jnp.zeros_like(o)
    for w, b in ((w1, b1), (w2, b2), (w3, b3)):
        y = (jnp.dot(agg, w[...], preferred_element_type=jnp.float32)
             + deg * b[...])
        acc += jnp.maximum(y, 0.0)
    o[...] = acc / jnp.maximum(deg, 1.0)


@jax.jit
def _tc_dense(agg, degp, W1, b1, W2, b2, W3, b3):
    BR = 1000
    grid = (N // BR,)
    wspec = pl.BlockSpec((D, H), lambda i: (0, 0))
    bspec = pl.BlockSpec((1, H), lambda i: (0, 0))
    return pl.pallas_call(
        _tc_body,
        grid=grid,
        in_specs=[
            pl.BlockSpec((BR, D), lambda i: (i, 0)),
            pl.BlockSpec((BR, D), lambda i: (i + N // BR, 0)),
            pl.BlockSpec((BR, NW), lambda i: (i, 0)),
            wspec, bspec, wspec, bspec, wspec, bspec,
        ],
        out_specs=pl.BlockSpec((BR, H), lambda i: (i, 0)),
        out_shape=jax.ShapeDtypeStruct((N, H), jnp.float32),
    )(agg, agg, degp, W1, b1, W2, b2, W3, b3)


def kernel(x, edge_index, batch, W1, b1, W2, b2, W3, b3):
    src = edge_index[0]
    dst = edge_index[1]
    agg, degp = _sc_aggregate(x, src, dst)
    return _tc_dense(agg, degp.reshape(NW, N).T, W1, b1.reshape(1, H),
                     W2, b2.reshape(1, H), W3, b3.reshape(1, H))


# speedup vs baseline: 33.8468x; 1.1109x over previous
import jax
import jax.numpy as jnp
from jax import lax
from jax.experimental import pallas as pl
from jax.experimental.pallas import tpu as pltpu
from jax.experimental.pallas import tpu_sc as plsc

N = 10000
E = 320000
D = 128
H = 128

NS = 16
K = 128
NROWS = E // K
NC = 2
NW = NC * NS
CMAX = NROWS // NW + 1
IB = 15
NBLK = 6
STRIPE = 624
TAIL = N - NS * STRIPE


def _sc_body(x_hbm, ei_hbm, agg_out, degp_out,
             src_a, dst_a, src_b, dst_b, dst_stage, rows0, rows1,
             deg_local, agg_sh, sem0, sem1, semi):
    cid = lax.axis_index("c")
    sid = lax.axis_index("s")
    wid = sid * NC + cid

    zv = jnp.zeros((16,), jnp.float32)

    def _zero_rows(i, _):
        rows0[i // 8, pl.ds((i % 8) * 16, 16)] = zv
        return 0
    lax.fori_loop(0, K * D // 16, _zero_rows, 0)

    def _zero_deg(i, _):
        deg_local[0, pl.ds(i * 16, 16)] = zv
        return 0
    lax.fori_loop(0, N // 16, _zero_deg, 0)

    base = sid * STRIPE
    off = 0
    for sz in (128, 128, 128, 128, 112):
        pltpu.sync_copy(rows0.at[pl.ds(0, sz)], agg_sh.at[pl.ds(base + off, sz)])
        off += sz

    @pl.when(sid == NS - 1)
    def _zero_tail():
        pltpu.sync_copy(rows0.at[pl.ds(0, TAIL)],
                        agg_sh.at[pl.ds(NS * STRIPE, TAIL)])

    plsc.subcore_barrier()

    c0 = (CMAX - 1) * wid + jnp.minimum(wid, NROWS - (CMAX - 1) * NW)
    nc = jnp.where(wid < NROWS - (CMAX - 1) * NW, CMAX, CMAX - 1)
    cnt5 = nc - (NBLK - 1) * IB

    ones_v = jnp.full((16,), 1.0, jnp.float32)
    zeros_i = jnp.zeros((16,), jnp.int32)
    rows_bufs = (rows0, rows1)
    sems = (sem0, sem1)
    idx_bufs = ((src_a, dst_a), (src_b, dst_b))

    def _load_idx(bi, bufp, size, sync):
        sbuf, dbuf = idx_bufs[bufp]
        bc = (c0 + bi * IB) * K
        if sync:
            pltpu.sync_copy(ei_hbm.at[pl.ds(bc, size * K)],
                            sbuf.at[pl.ds(0, size * K)])
            pltpu.sync_copy(ei_hbm.at[pl.ds(E + bc, size * K)],
                            dbuf.at[pl.ds(0, size * K)])
        else:
            pltpu.async_copy(ei_hbm.at[pl.ds(bc, size * K)],
                             sbuf.at[pl.ds(0, size * K)], semi)
            pltpu.async_copy(ei_hbm.at[pl.ds(E + bc, size * K)],
                             dbuf.at[pl.ds(0, size * K)], semi)

    def _wait_idx(bufp, size):
        sbuf, dbuf = idx_bufs[bufp]
        bc = c0 * K
        pltpu.make_async_copy(ei_hbm.at[pl.ds(bc, size * K)],
                              sbuf.at[pl.ds(0, size * K)], semi).wait()
        pltpu.make_async_copy(ei_hbm.at[pl.ds(bc, size * K)],
                              dbuf.at[pl.ds(0, size * K)], semi).wait()

    def _prefetch(bi):
        if bi < NBLK - 1:
            _load_idx(bi, bi % 2, IB, sync=False)
        else:
            @pl.when(wid < NW - 1)
            def _pf_full():
                _load_idx(bi, bi % 2, IB, sync=False)

            @pl.when(wid == NW - 1)
            def _pf_part():
                _load_idx(bi, bi % 2, 3, sync=False)

    def _wait_prefetch(bi):
        if bi < NBLK - 1:
            _wait_idx(bi % 2, IB)
        else:
            @pl.when(wid < NW - 1)
            def _wf_full():
                _wait_idx(bi % 2, IB)

            @pl.when(wid == NW - 1)
            def _wf_part():
                _wait_idx(bi % 2, 3)

    def _fire(bufp, j, rp):
        sbuf = idx_bufs[bufp][0]
        pltpu.async_copy(x_hbm.at[sbuf.at[pl.ds(j * K, K)]],
                         rows_bufs[rp], sems[rp])

    def _consume(bufp, j, rp):
        dbuf = idx_bufs[bufp][1]
        for l in range(K // 16):
            d16 = dbuf[pl.ds(j * K + l * 16, 16)]
            dst_stage[0, pl.ds(l * 16, 16)] = d16
            plsc.addupdate_scatter(deg_local, [zeros_i, d16], ones_v)
        pltpu.make_async_copy(x_hbm.at[idx_bufs[0][0].at[pl.ds(0, K)]],
                              rows_bufs[rp], sems[rp]).wait()
        pltpu.sync_copy(rows_bufs[rp], agg_sh.at[dst_stage.at[0]], add=True)

    _load_idx(0, 0, IB, sync=True)
    _fire(0, 0, 0)
    _prefetch(1)

    for b in range(NBLK):
        bufp = b % 2
        blk_cnt = IB if b < NBLK - 1 else 4
        for j in range(blk_cnt):
            rp = (j + b) % 2
            guard = (b == NBLK - 1) and (j >= 3)
            if j + 1 < blk_cnt:
                if (b == NBLK - 1) and (j + 1 >= 3):
                    @pl.when(j + 1 < cnt5)
                    def _fire_nextg():
                        _fire(bufp, j + 1, 1 - rp)
                else:
                    _fire(bufp, j + 1, 1 - rp)
            elif b < NBLK - 1:
                _wait_prefetch(b + 1)
                _fire(1 - bufp, 0, 1 - rp)
            if guard:
                @pl.when(j < cnt5)
                def _consume_g():
                    _consume(bufp, j, rp)
            else:
                _consume(bufp, j, rp)
        if b + 2 < NBLK:
            _prefetch(b + 2)
    plsc.subcore_barrier()

    pltpu.sync_copy(agg_sh.at[pl.ds(base, STRIPE)],
                    agg_out.at[pl.ds(cid * N + base, STRIPE)])

    @pl.when(sid == NS - 1)
    def _write_tail():
        pltpu.sync_copy(agg_sh.at[pl.ds(NS * STRIPE, TAIL)],
                        agg_out.at[pl.ds(cid * N + NS * STRIPE, TAIL)])

    pltpu.sync_copy(deg_local.at[0], degp_out.at[pl.ds(wid * N, N)])


@jax.jit
def _sc_aggregate(x, ei):
    mesh = plsc.VectorSubcoreMesh(core_axis_name="c", subcore_axis_name="s")
    f = pl.kernel(
        _sc_body,
        out_type=[
            jax.ShapeDtypeStruct((NC * N, D), jnp.float32),
            jax.ShapeDtypeStruct((NW * N,), jnp.float32),
        ],
        mesh=mesh,
        compiler_params=pltpu.CompilerParams(needs_layout_passes=False),
        scratch_types=[
            pltpu.VMEM((IB * K,), jnp.int32),
            pltpu.VMEM((IB * K,), jnp.int32),
            pltpu.VMEM((IB * K,), jnp.int32),
            pltpu.VMEM((IB * K,), jnp.int32),
            pltpu.VMEM((1, K), jnp.int32),
            pltpu.VMEM((K, D), jnp.float32),
            pltpu.VMEM((K, D), jnp.float32),
            pltpu.VMEM((1, N), jnp.float32),
            pltpu.VMEM_SHARED((N, D), jnp.float32),
            pltpu.SemaphoreType.DMA,
            pltpu.SemaphoreType.DMA,
            pltpu.SemaphoreType.DMA,
        ],
    )
    return f(x, ei)


def _tc_body(a0, a1, dp, w1, b1, w2, b2, w3, b3, o):
    deg = jnp.sum(dp[...], axis=1, keepdims=True)
    agg = a0[...] + a1[...]
    acc = jnp.zeros_like(o)
    for w, b in ((w1, b1), (w2, b2), (w3, b3)):
        y = (jnp.dot(agg, w[...], preferred_element_type=jnp.float32)
             + deg * b[...])
        acc += jnp.maximum(y, 0.0)
    o[...] = acc / jnp.maximum(deg, 1.0)


@jax.jit
def _tc_dense(agg, degp, W1, b1, W2, b2, W3, b3):
    BR = 1000
    grid = (N // BR,)
    wspec = pl.BlockSpec((D, H), lambda i: (0, 0))
    bspec = pl.BlockSpec((1, H), lambda i: (0, 0))
    return pl.pallas_call(
        _tc_body,
        grid=grid,
        in_specs=[
            pl.BlockSpec((BR, D), lambda i: (i, 0)),
            pl.BlockSpec((BR, D), lambda i: (i + N // BR, 0)),
            pl.BlockSpec((BR, NW), lambda i: (i, 0)),
            wspec, bspec, wspec, bspec, wspec, bspec,
        ],
        out_specs=pl.BlockSpec((BR, H), lambda i: (i, 0)),
        out_shape=jax.ShapeDtypeStruct((N, H), jnp.float32),
    )(agg, agg, degp, W1, b1, W2, b2, W3, b3)


def kernel(x, edge_index, batch, W1, b1, W2, b2, W3, b3):
    agg, degp = _sc_aggregate(x, edge_index.reshape(2 * E))
    return _tc_dense(agg, degp.reshape(NW, N).T, W1, b1.reshape(1, H),
                     W2, b2.reshape(1, H), W3, b3.reshape(1, H))


# speedup vs baseline: 34.5016x; 1.0193x over previous
import jax
import jax.numpy as jnp
from jax import lax
from jax.experimental import pallas as pl
from jax.experimental.pallas import tpu as pltpu
from jax.experimental.pallas import tpu_sc as plsc

N = 10000
E = 320000
D = 128
H = 128

NS = 16
K = 128
NROWS = E // K
NC = 2
NW = NC * NS
CMAX = NROWS // NW + 1
IB = 15
NBLK = 6
STRIPE = 624
TAIL = N - NS * STRIPE


def _sc_body(x_hbm, ei_hbm, agg_out, degp_out,
             src_a, dst_a, src_b, dst_b, dst_stage, rows0, rows1,
             deg_local, agg_sh, sem0, sem1, semi):
    cid = lax.axis_index("c")
    sid = lax.axis_index("s")
    wid = sid * NC + cid

    zv = jnp.zeros((16,), jnp.float32)

    def _zero_rows(i, _):
        rows0[i // 8, pl.ds((i % 8) * 16, 16)] = zv
        return 0
    lax.fori_loop(0, K * D // 16, _zero_rows, 0)

    def _zero_deg(i, _):
        deg_local[0, pl.ds(i * 16, 16)] = zv
        return 0
    lax.fori_loop(0, N // 16, _zero_deg, 0)

    base = sid * STRIPE
    off = 0
    for sz in (128, 128, 128, 128, 112):
        pltpu.sync_copy(rows0.at[pl.ds(0, sz)], agg_sh.at[pl.ds(base + off, sz)])
        off += sz

    @pl.when(sid == NS - 1)
    def _zero_tail():
        pltpu.sync_copy(rows0.at[pl.ds(0, TAIL)],
                        agg_sh.at[pl.ds(NS * STRIPE, TAIL)])

    plsc.subcore_barrier()

    c0 = (CMAX - 1) * wid + jnp.minimum(wid, NROWS - (CMAX - 1) * NW)
    nc = jnp.where(wid < NROWS - (CMAX - 1) * NW, CMAX, CMAX - 1)
    cnt5 = nc - (NBLK - 1) * IB

    ones_v = jnp.full((16,), 1.0, jnp.float32)
    zeros_i = jnp.zeros((16,), jnp.int32)
    rows_bufs = (rows0, rows1)
    sems = (sem0, sem1)
    idx_bufs = ((src_a, dst_a), (src_b, dst_b))

    def _load_idx(bi, bufp, size, sync):
        sbuf, dbuf = idx_bufs[bufp]
        bc = (c0 + bi * IB) * K
        if sync:
            pltpu.sync_copy(ei_hbm.at[pl.ds(bc, size * K)],
                            sbuf.at[pl.ds(0, size * K)])
            pltpu.sync_copy(ei_hbm.at[pl.ds(E + bc, size * K)],
                            dbuf.at[pl.ds(0, size * K)])
        else:
            pltpu.async_copy(ei_hbm.at[pl.ds(bc, size * K)],
                             sbuf.at[pl.ds(0, size * K)], semi)
            pltpu.async_copy(ei_hbm.at[pl.ds(E + bc, size * K)],
                             dbuf.at[pl.ds(0, size * K)], semi)

    def _wait_idx(bufp, size):
        sbuf, dbuf = idx_bufs[bufp]
        bc = c0 * K
        pltpu.make_async_copy(ei_hbm.at[pl.ds(bc, size * K)],
                              sbuf.at[pl.ds(0, size * K)], semi).wait()
        pltpu.make_async_copy(ei_hbm.at[pl.ds(bc, size * K)],
                              dbuf.at[pl.ds(0, size * K)], semi).wait()

    def _prefetch(bi):
        if bi < NBLK - 1:
            _load_idx(bi, bi % 2, IB, sync=False)
        else:
            @pl.when(wid < NW - 1)
            def _pf_full():
                _load_idx(bi, bi % 2, IB, sync=False)

            @pl.when(wid == NW - 1)
            def _pf_part():
                _load_idx(bi, bi % 2, 3, sync=False)

    def _wait_prefetch(bi):
        if bi < NBLK - 1:
            _wait_idx(bi % 2, IB)
        else:
            @pl.when(wid < NW - 1)
            def _wf_full():
                _wait_idx(bi % 2, IB)

            @pl.when(wid == NW - 1)
            def _wf_part():
                _wait_idx(bi % 2, 3)

    def _fire(bufp, j, rp):
        sbuf = idx_bufs[bufp][0]
        pltpu.async_copy(x_hbm.at[sbuf.at[pl.ds(j * K, K)]],
                         rows_bufs[rp], sems[rp])

    def _consume(bufp, j, rp):
        dbuf = idx_bufs[bufp][1]
        for l in range(K // 16):
            d16 = dbuf[pl.ds(j * K + l * 16, 16)]
            dst_stage[0, pl.ds(l * 16, 16)] = d16
            plsc.addupdate_scatter(deg_local, [zeros_i, d16], ones_v)
        pltpu.make_async_copy(x_hbm.at[idx_bufs[0][0].at[pl.ds(0, K)]],
                              rows_bufs[rp], sems[rp]).wait()
        pltpu.sync_copy(rows_bufs[rp], agg_sh.at[dst_stage.at[0]], add=True)

    _load_idx(0, 0, IB, sync=True)
    _fire(0, 0, 0)
    _prefetch(1)

    for b in range(NBLK):
        bufp = b % 2
        blk_cnt = IB if b < NBLK - 1 else 4
        for j in range(blk_cnt):
            rp = (j + b) % 2
            guard = (b == NBLK - 1) and (j >= 3)
            if j + 1 < blk_cnt:
                if (b == NBLK - 1) and (j + 1 >= 3):
                    @pl.when(j + 1 < cnt5)
                    def _fire_nextg():
                        _fire(bufp, j + 1, 1 - rp)
                else:
                    _fire(bufp, j + 1, 1 - rp)
            elif b < NBLK - 1:
                _wait_prefetch(b + 1)
                _fire(1 - bufp, 0, 1 - rp)
            if guard:
                @pl.when(j < cnt5)
                def _consume_g():
                    _consume(bufp, j, rp)
            else:
                _consume(bufp, j, rp)
        if b + 2 < NBLK:
            _prefetch(b + 2)
    plsc.subcore_barrier()

    pltpu.sync_copy(agg_sh.at[pl.ds(base, STRIPE)],
                    agg_out.at[pl.ds(cid * N + base, STRIPE)])

    @pl.when(sid == NS - 1)
    def _write_tail():
        pltpu.sync_copy(agg_sh.at[pl.ds(NS * STRIPE, TAIL)],
                        agg_out.at[pl.ds(cid * N + NS * STRIPE, TAIL)])

    pltpu.sync_copy(deg_local.at[0], degp_out.at[pl.ds(wid * N, N)])


@jax.jit
def _sc_aggregate(x, ei):
    mesh = plsc.VectorSubcoreMesh(core_axis_name="c", subcore_axis_name="s")
    f = pl.kernel(
        _sc_body,
        out_type=[
            jax.ShapeDtypeStruct((NC * N, D), jnp.float32),
            jax.ShapeDtypeStruct((NW * N,), jnp.float32),
        ],
        mesh=mesh,
        compiler_params=pltpu.CompilerParams(needs_layout_passes=False),
        scratch_types=[
            pltpu.VMEM((IB * K,), jnp.int32),
            pltpu.VMEM((IB * K,), jnp.int32),
            pltpu.VMEM((IB * K,), jnp.int32),
            pltpu.VMEM((IB * K,), jnp.int32),
            pltpu.VMEM((1, K), jnp.int32),
            pltpu.VMEM((K, D), jnp.float32),
            pltpu.VMEM((K, D), jnp.float32),
            pltpu.VMEM((1, N), jnp.float32),
            pltpu.VMEM_SHARED((N, D), jnp.float32),
            pltpu.SemaphoreType.DMA,
            pltpu.SemaphoreType.DMA,
            pltpu.SemaphoreType.DMA,
        ],
    )
    return f(x, ei)


def _tc_body(a0, a1, dp, w1, b1, w2, b2, w3, b3, o):
    deg = jnp.sum(dp[...], axis=1, keepdims=True)
    agg = a0[...] + a1[...]
    acc = jnp.zeros_like(o)
    for w, b in ((w1, b1), (w2, b2), (w3, b3)):
        y = (jnp.dot(agg, w[...], preferred_element_type=jnp.float32)
             + deg * b[...])
        acc += jnp.maximum(y, 0.0)
    o[...] = acc / jnp.maximum(deg, 1.0)


@jax.jit
def _tc_dense(agg, degp, W1, b1, W2, b2, W3, b3):
    BR = 2000
    grid = (N // BR,)
    wspec = pl.BlockSpec((D, H), lambda i: (0, 0))
    bspec = pl.BlockSpec((1, H), lambda i: (0, 0))
    return pl.pallas_call(
        _tc_body,
        grid=grid,
        in_specs=[
            pl.BlockSpec((BR, D), lambda i: (i, 0)),
            pl.BlockSpec((BR, D), lambda i: (i + N // BR, 0)),
            pl.BlockSpec((BR, NW), lambda i: (i, 0)),
            wspec, bspec, wspec, bspec, wspec, bspec,
        ],
        out_specs=pl.BlockSpec((BR, H), lambda i: (i, 0)),
        out_shape=jax.ShapeDtypeStruct((N, H), jnp.float32),
    )(agg, agg, degp, W1, b1, W2, b2, W3, b3)


def kernel(x, edge_index, batch, W1, b1, W2, b2, W3, b3):
    agg, degp = _sc_aggregate(x, edge_index.reshape(2 * E))
    return _tc_dense(agg, degp.reshape(NW, N).T, W1, b1.reshape(1, H),
                     W2, b2.reshape(1, H), W3, b3.reshape(1, H))
